# baseline (speedup 1.0000x reference)
import functools

import jax
import jax.numpy as jnp
from jax import lax
from jax.experimental import pallas as pl
from jax.experimental.pallas import tpu as pltpu
from jax.experimental.pallas import tpu_sc as plsc

N = 10000
D = 128
E = 320000
NC, NS = 2, 16
NW = NC * NS
CH = 64
E_PAD = 327680
PER_W = E_PAD // NW
NCHUNK = PER_W // CH
N_PAD = 10240
TROWS = N_PAD // NS
DEG_PAD = 10240
DROWS = DEG_PAD // NS

f32 = jnp.float32


def _deg_body(col_hbm, w_hbm, out_hbm, cidx, wsl, zbuf, deg_s):
    sc = lax.axis_index("c")
    tid = lax.axis_index("s")
    wid = tid * NC + sc

    def zb(i, carry):
        zbuf[pl.ds(i * 16, 16)] = jnp.zeros((16,), f32)
        return carry

    lax.fori_loop(0, DROWS // 16, zb, 0)
    pltpu.sync_copy(zbuf, deg_s.at[pl.ds(tid * DROWS, DROWS)])
    plsc.subcore_barrier()

    def chunk(i, carry):
        off = wid * PER_W + i * CH
        pltpu.sync_copy(col_hbm.at[pl.ds(off, CH)], cidx.at[0])
        pltpu.sync_copy(w_hbm.at[pl.ds(off, CH)], wsl.at[0])
        pltpu.sync_copy(wsl.at[0], deg_s.at[cidx.at[0]], add=True)
        return carry

    lax.fori_loop(0, NCHUNK, chunk, 0)
    plsc.subcore_barrier()
    pltpu.sync_copy(deg_s.at[pl.ds(tid * DROWS, DROWS)],
                    out_hbm.at[sc, pl.ds(tid * DROWS, DROWS)])


NBUF = 5
NGRP = NCHUNK // NBUF


def _prop_body(g_hbm, epk_hbm, wck_hbm, out_hbm, eidx, wsl, rows, acc_s,
               *sems):
    ids, gat, sct = sems[:NBUF], sems[NBUF:2 * NBUF], sems[2 * NBUF:]
    sc = lax.axis_index("c")
    tid = lax.axis_index("s")
    wid = tid * NC + sc
    cbase = wid * NCHUNK

    def zrow(i, carry):
        for j in range(D // 16):
            rows[0, i, pl.ds(j * 16, 16)] = jnp.zeros((16,), f32)
        return carry

    lax.fori_loop(0, CH, zrow, 0)
    base = tid * TROWS
    for off in range(0, TROWS, CH):
        pltpu.sync_copy(rows.at[0], acc_s.at[pl.ds(base + off, CH)])
    plsc.subcore_barrier()

    def idx_start(b, c):
        pltpu.async_copy(epk_hbm.at[c], eidx.at[b], ids[b])
        pltpu.async_copy(wck_hbm.at[c], wsl.at[b], ids[b])

    def idx_wait(b, c):
        pltpu.make_async_copy(epk_hbm.at[c], eidx.at[b], ids[b]).wait()
        pltpu.make_async_copy(wck_hbm.at[c], wsl.at[b], ids[b]).wait()

    def gat_start(b):
        pltpu.async_copy(g_hbm.at[eidx.at[b, 0]], rows.at[b], gat[b])

    def gat_wait(b):
        pltpu.make_async_copy(g_hbm.at[eidx.at[b, 0]], rows.at[b],
                              gat[b]).wait()

    def sct_start(b):
        pltpu.async_copy(rows.at[b], acc_s.at[eidx.at[b, 1]], sct[b],
                         add=True)

    def sct_wait(b):
        pltpu.make_async_copy(rows.at[b], acc_s.at[eidx.at[b, 1]],
                              sct[b]).wait()

    def scale(b):
        def group(gi, c2):
            wv = wsl[b, pl.ds(gi * 16, 16)]
            for lane in range(16):
                s = wv[lane]
                e = gi * 16 + lane
                for j in range(D // 16):
                    sl = pl.ds(j * 16, 16)
                    rows[b, e, sl] = rows[b, e, sl] * s
            return c2

        lax.fori_loop(0, CH // 16, group, 0)

    for b in range(NBUF):
        idx_start(b, cbase + b)
    for b in range(NBUF):
        idx_wait(b, cbase + b)
        gat_start(b)

    def grp(g, carry):
        for b in range(NBUF):
            gat_wait(b)
            scale(b)
            sct_start(b)
        for b in range(NBUF):
            c_next = cbase + lax.rem(g * NBUF + b + NBUF, NCHUNK)
            sct_wait(b)
            idx_start(b, c_next)
        for b in range(NBUF):
            c_next = cbase + lax.rem(g * NBUF + b + NBUF, NCHUNK)
            idx_wait(b, c_next)
            gat_start(b)
        return carry

    lax.fori_loop(0, NGRP, grp, 0)
    for b in range(NBUF):
        gat_wait(b)
    plsc.subcore_barrier()
    pltpu.sync_copy(acc_s.at[pl.ds(base, TROWS)],
                    out_hbm.at[sc, pl.ds(base, TROWS)])


@functools.cache
def _sc_kernels():
    mesh = plsc.VectorSubcoreMesh(core_axis_name="c", subcore_axis_name="s")
    deg = functools.partial(
        pl.kernel,
        out_type=jax.ShapeDtypeStruct((NC, DEG_PAD), f32),
        mesh=mesh,
        scratch_types=[
            pltpu.VMEM((1, CH), jnp.int32),
            pltpu.VMEM((1, CH), f32),
            pltpu.VMEM((DROWS,), f32),
            pltpu.VMEM_SHARED((DEG_PAD,), f32),
        ],
    )(_deg_body)
    prop = functools.partial(
        pl.kernel,
        out_type=jax.ShapeDtypeStruct((NC, N_PAD, D), f32),
        mesh=mesh,
        scratch_types=[
            pltpu.VMEM((NBUF, 2, CH), jnp.int32),
            pltpu.VMEM((NBUF, CH), f32),
            pltpu.VMEM((NBUF, CH, D), f32),
            pltpu.VMEM_SHARED((N_PAD, D), f32),
        ] + [pltpu.SemaphoreType.DMA] * (3 * NBUF),
    )(_prop_body)
    return deg, prop


BT = 1000


def _spec_x(i):
    return (i, 0)


def _spec_acc(i):
    return (0, i, 0)


def _spec_w(i):
    return (0, 0)


_X = pl.BlockSpec((BT, D), _spec_x)
_ACC = pl.BlockSpec((NC, BT, D), _spec_acc)
_DINV = pl.BlockSpec((BT, 1), _spec_x)
_DEG = pl.BlockSpec((NC, BT, 1), _spec_acc)
_W = pl.BlockSpec((D, D), _spec_w)
_B = pl.BlockSpec((8, D), _spec_w)
_GRID = N // BT


def _prep_body(x_ref, deg_ref, w_ref, dinv_ref, g_ref, s_ref):
    d = deg_ref[0] + deg_ref[1]
    dinv = jnp.where(d > 0, lax.rsqrt(jnp.where(d > 0, d, 1.0)), 0.0)
    dinv_ref[...] = dinv
    g_ref[...] = x_ref[...] * dinv
    s_ref[...] = jnp.dot(x_ref[...], w_ref[...], preferred_element_type=f32)


_prep = pl.pallas_call(
    _prep_body,
    grid=(_GRID,),
    in_specs=[_X, _DEG, _W],
    out_specs=[_DINV, _X, _X],
    out_shape=[jax.ShapeDtypeStruct((N, 1), f32),
               jax.ShapeDtypeStruct((N, D), f32),
               jax.ShapeDtypeStruct((N, D), f32)],
)


def _step_body(acc_ref, dinv_ref, sin_ref, w_ref, sout_ref, g_ref):
    dinv = dinv_ref[...]
    h = (acc_ref[0] + acc_ref[1]) * dinv
    sout_ref[...] = sin_ref[...] + jnp.dot(h, w_ref[...],
                                           preferred_element_type=f32)
    g_ref[...] = h * dinv


_step = pl.pallas_call(
    _step_body,
    grid=(_GRID,),
    in_specs=[_ACC, _DINV, _X, _W],
    out_specs=[_X, _X],
    out_shape=[jax.ShapeDtypeStruct((N, D), f32),
               jax.ShapeDtypeStruct((N, D), f32)],
)


def _layerend_body(acc_ref, dinv_ref, sin_ref, w_ref, b_ref, wn_ref,
                   g_ref, snext_ref):
    dinv = dinv_ref[...]
    h = (acc_ref[0] + acc_ref[1]) * dinv
    s = sin_ref[...] + jnp.dot(h, w_ref[...], preferred_element_type=f32)
    s = s + b_ref[0:1, :]
    h0 = jnp.where(s > 0, s, jnp.exp(jnp.minimum(s, 0.0)) - 1.0)
    g_ref[...] = h0 * dinv
    snext_ref[...] = jnp.dot(h0, wn_ref[...], preferred_element_type=f32)


_layerend = pl.pallas_call(
    _layerend_body,
    grid=(_GRID,),
    in_specs=[_ACC, _DINV, _X, _W, _B, _W],
    out_specs=[_X, _X],
    out_shape=[jax.ShapeDtypeStruct((N, D), f32),
               jax.ShapeDtypeStruct((N, D), f32)],
)


def _final_body(acc_ref, dinv_ref, sin_ref, w_ref, b_ref, out_ref):
    h = (acc_ref[0] + acc_ref[1]) * dinv_ref[...]
    out_ref[...] = (sin_ref[...]
                    + jnp.dot(h, w_ref[...], preferred_element_type=f32)
                    + b_ref[0:1, :])


_final = pl.pallas_call(
    _final_body,
    grid=(_GRID,),
    in_specs=[_ACC, _DINV, _X, _W, _B],
    out_specs=_X,
    out_shape=jax.ShapeDtypeStruct((N, D), f32),
)


def kernel(x, edge_index, weight, W1, b1, W2, b2, W3, b3):
    row = edge_index[0].astype(jnp.int32)
    col = edge_index[1].astype(jnp.int32)
    w = weight.astype(f32)
    pad = E_PAD - E
    pidx = (jnp.arange(pad, dtype=jnp.int32) * 131) % N
    rowp = jnp.concatenate([row, pidx])
    colp = jnp.concatenate([col, pidx])
    wp = jnp.concatenate([w, jnp.zeros((pad,), f32)])
    epk = (jnp.stack([rowp, colp], axis=0)
           .reshape(2, E_PAD // CH, CH).transpose(1, 0, 2))
    wck = wp.reshape(E_PAD // CH, CH)

    deg_kernel, prop_kernel = _sc_kernels()
    deg2 = deg_kernel(colp, wp)[:, :N].reshape(NC, N, 1)

    Ws = (W1, W2, W3)
    bs = (jnp.broadcast_to(b1, (8, D)), jnp.broadcast_to(b2, (8, D)),
          jnp.broadcast_to(b3, (8, D)))
    dinv, g, s = _prep(x, deg2, W1[0])
    out = None
    for l in range(3):
        for k in range(1, 4):
            acc2 = prop_kernel(g, epk, wck)[:, :N]
            if k < 3:
                s, g = _step(acc2, dinv, s, Ws[l][k])
            elif l < 2:
                g, s = _layerend(acc2, dinv, s, Ws[l][3], bs[l], Ws[l + 1][0])
            else:
                out = _final(acc2, dinv, s, Ws[2][3], bs[2])
    return out


# speedup vs baseline: 1.1299x; 1.1299x over previous
import functools

import jax
import jax.numpy as jnp
from jax import lax
from jax.experimental import pallas as pl
from jax.experimental.pallas import tpu as pltpu
from jax.experimental.pallas import tpu_sc as plsc

N = 10000
D = 128
E = 320000
NC, NS = 2, 16
NW = NC * NS
CH = 128
E_PAD = 327680
PER_W = E_PAD // NW
NCHUNK = PER_W // CH
ZR = 632
ZR_LAST = N - (NS - 1) * ZR
WSCALE = float(2 ** 23)
DEG_PAD = 10240
DROWS = DEG_PAD // NS

f32 = jnp.float32


def _deg_body(col_hbm, w_hbm, out_hbm, cidx, wsl, zbuf, deg_s):
    sc = lax.axis_index("c")
    tid = lax.axis_index("s")
    wid = tid * NC + sc

    def zb(i, carry):
        zbuf[pl.ds(i * 16, 16)] = jnp.zeros((16,), f32)
        return carry

    lax.fori_loop(0, DROWS // 16, zb, 0)
    pltpu.sync_copy(zbuf, deg_s.at[pl.ds(tid * DROWS, DROWS)])
    plsc.subcore_barrier()

    def chunk(i, carry):
        off = wid * PER_W + i * CH
        pltpu.sync_copy(col_hbm.at[pl.ds(off, CH)], cidx.at[0])
        pltpu.sync_copy(w_hbm.at[pl.ds(off, CH)], wsl.at[0])
        pltpu.sync_copy(wsl.at[0], deg_s.at[cidx.at[0]], add=True)
        return carry

    lax.fori_loop(0, NCHUNK, chunk, 0)
    plsc.subcore_barrier()
    pltpu.sync_copy(deg_s.at[pl.ds(tid * DROWS, DROWS)],
                    out_hbm.at[sc, pl.ds(tid * DROWS, DROWS)])


NBUF = 3
NGRP = NCHUNK // NBUF
NTAIL = NCHUNK - NGRP * NBUF


def _prop_body(g_hbm, epk_hbm, out_hbm, eidx, rows, acc_s, *sems):
    ids, gat, sct = sems[:NBUF], sems[NBUF:2 * NBUF], sems[2 * NBUF:]
    sc = lax.axis_index("c")
    tid = lax.axis_index("s")
    wid = tid * NC + sc
    cbase = wid * NCHUNK

    def zrow(i, carry):
        for j in range(D // 16):
            rows[0, i, pl.ds(j * 16, 16)] = jnp.zeros((16,), f32)
        return carry

    lax.fori_loop(0, CH, zrow, 0)
    base = tid * ZR

    def zero_range(nrows):
        for off in range(0, nrows, CH):
            m = min(CH, nrows - off)
            pltpu.sync_copy(rows.at[0, pl.ds(0, m)],
                            acc_s.at[pl.ds(base + off, m)])

    @pl.when(tid < NS - 1)
    def _():
        zero_range(ZR)

    @pl.when(tid == NS - 1)
    def _():
        zero_range(ZR_LAST)

    plsc.subcore_barrier()

    def idx_start(b, c):
        pltpu.async_copy(epk_hbm.at[c], eidx.at[b], ids[b])

    def idx_wait(b, c):
        pltpu.make_async_copy(epk_hbm.at[c], eidx.at[b], ids[b]).wait()

    def gat_start(b):
        pltpu.async_copy(g_hbm.at[eidx.at[b, 0]], rows.at[b], gat[b])

    def gat_wait(b):
        pltpu.make_async_copy(g_hbm.at[eidx.at[b, 0]], rows.at[b],
                              gat[b]).wait()

    def sct_start(b):
        pltpu.async_copy(rows.at[b], acc_s.at[eidx.at[b, 1]], sct[b],
                         add=True)

    def sct_wait(b):
        pltpu.make_async_copy(rows.at[b], acc_s.at[eidx.at[b, 1]],
                              sct[b]).wait()

    def scale(b):
        def group(gi, c2):
            wv = (eidx[b, 2, pl.ds(gi * 16, 16)].astype(f32)
                  * jnp.float32(1.0 / WSCALE))
            for lane in range(16):
                s = wv[lane]
                e = gi * 16 + lane
                for j in range(D // 16):
                    sl = pl.ds(j * 16, 16)
                    rows[b, e, sl] = rows[b, e, sl] * s
            return c2

        lax.fori_loop(0, CH // 16, group, 0)

    for b in range(NBUF):
        idx_start(b, cbase + b)
    for b in range(NBUF):
        idx_wait(b, cbase + b)
        gat_start(b)

    def grp(g, carry):
        for b in range(NBUF):
            gat_wait(b)
            scale(b)
            sct_start(b)
        for b in range(NBUF):
            c_next = cbase + lax.rem(g * NBUF + b + NBUF, NCHUNK)
            sct_wait(b)
            idx_start(b, c_next)
        for b in range(NBUF):
            c_next = cbase + lax.rem(g * NBUF + b + NBUF, NCHUNK)
            idx_wait(b, c_next)
            gat_start(b)
        return carry

    lax.fori_loop(0, NGRP, grp, 0)
    for b in range(NTAIL):
        gat_wait(b)
        scale(b)
        sct_start(b)
    for b in range(NTAIL):
        sct_wait(b)
    for b in range(NTAIL, NBUF):
        gat_wait(b)
    plsc.subcore_barrier()

    @pl.when(tid < NS - 1)
    def _():
        pltpu.sync_copy(acc_s.at[pl.ds(base, ZR)],
                        out_hbm.at[sc, pl.ds(base, ZR)])

    @pl.when(tid == NS - 1)
    def _():
        pltpu.sync_copy(acc_s.at[pl.ds(base, ZR_LAST)],
                        out_hbm.at[sc, pl.ds(base, ZR_LAST)])


@functools.cache
def _sc_kernels():
    mesh = plsc.VectorSubcoreMesh(core_axis_name="c", subcore_axis_name="s")
    deg = functools.partial(
        pl.kernel,
        out_type=jax.ShapeDtypeStruct((NC, DEG_PAD), f32),
        mesh=mesh,
        scratch_types=[
            pltpu.VMEM((1, CH), jnp.int32),
            pltpu.VMEM((1, CH), f32),
            pltpu.VMEM((DROWS,), f32),
            pltpu.VMEM_SHARED((DEG_PAD,), f32),
        ],
    )(_deg_body)
    prop = functools.partial(
        pl.kernel,
        out_type=jax.ShapeDtypeStruct((NC, N, D), f32),
        mesh=mesh,
        scratch_types=[
            pltpu.VMEM((NBUF, 3, CH), jnp.int32),
            pltpu.VMEM((NBUF, CH, D), f32),
            pltpu.VMEM_SHARED((N, D), f32),
        ] + [pltpu.SemaphoreType.DMA] * (3 * NBUF),
    )(_prop_body)
    return deg, prop


BT = 1000


def _spec_x(i):
    return (i, 0)


def _spec_acc(i):
    return (0, i, 0)


def _spec_w(i):
    return (0, 0)


_X = pl.BlockSpec((BT, D), _spec_x)
_ACC = pl.BlockSpec((NC, BT, D), _spec_acc)
_DINV = pl.BlockSpec((BT, 1), _spec_x)
_DEG = pl.BlockSpec((NC, BT, 1), _spec_acc)
_W = pl.BlockSpec((D, D), _spec_w)
_B = pl.BlockSpec((8, D), _spec_w)
_GRID = N // BT


def _prep_body(x_ref, deg_ref, w_ref, dinv_ref, g_ref, s_ref):
    d = deg_ref[0] + deg_ref[1]
    dinv = jnp.where(d > 0, lax.rsqrt(jnp.where(d > 0, d, 1.0)), 0.0)
    dinv_ref[...] = dinv
    g_ref[...] = x_ref[...] * dinv
    s_ref[...] = jnp.dot(x_ref[...], w_ref[...], preferred_element_type=f32)


_prep = pl.pallas_call(
    _prep_body,
    grid=(_GRID,),
    in_specs=[_X, _DEG, _W],
    out_specs=[_DINV, _X, _X],
    out_shape=[jax.ShapeDtypeStruct((N, 1), f32),
               jax.ShapeDtypeStruct((N, D), f32),
               jax.ShapeDtypeStruct((N, D), f32)],
)


def _step_body(acc_ref, dinv_ref, sin_ref, w_ref, sout_ref, g_ref):
    dinv = dinv_ref[...]
    h = (acc_ref[0] + acc_ref[1]) * dinv
    sout_ref[...] = sin_ref[...] + jnp.dot(h, w_ref[...],
                                           preferred_element_type=f32)
    g_ref[...] = h * dinv


_step = pl.pallas_call(
    _step_body,
    grid=(_GRID,),
    in_specs=[_ACC, _DINV, _X, _W],
    out_specs=[_X, _X],
    out_shape=[jax.ShapeDtypeStruct((N, D), f32),
               jax.ShapeDtypeStruct((N, D), f32)],
)


def _layerend_body(acc_ref, dinv_ref, sin_ref, w_ref, b_ref, wn_ref,
                   g_ref, snext_ref):
    dinv = dinv_ref[...]
    h = (acc_ref[0] + acc_ref[1]) * dinv
    s = sin_ref[...] + jnp.dot(h, w_ref[...], preferred_element_type=f32)
    s = s + b_ref[0:1, :]
    h0 = jnp.where(s > 0, s, jnp.exp(jnp.minimum(s, 0.0)) - 1.0)
    g_ref[...] = h0 * dinv
    snext_ref[...] = jnp.dot(h0, wn_ref[...], preferred_element_type=f32)


_layerend = pl.pallas_call(
    _layerend_body,
    grid=(_GRID,),
    in_specs=[_ACC, _DINV, _X, _W, _B, _W],
    out_specs=[_X, _X],
    out_shape=[jax.ShapeDtypeStruct((N, D), f32),
               jax.ShapeDtypeStruct((N, D), f32)],
)


def _final_body(acc_ref, dinv_ref, sin_ref, w_ref, b_ref, out_ref):
    h = (acc_ref[0] + acc_ref[1]) * dinv_ref[...]
    out_ref[...] = (sin_ref[...]
                    + jnp.dot(h, w_ref[...], preferred_element_type=f32)
                    + b_ref[0:1, :])


_final = pl.pallas_call(
    _final_body,
    grid=(_GRID,),
    in_specs=[_ACC, _DINV, _X, _W, _B],
    out_specs=_X,
    out_shape=jax.ShapeDtypeStruct((N, D), f32),
)


def kernel(x, edge_index, weight, W1, b1, W2, b2, W3, b3):
    row = edge_index[0].astype(jnp.int32)
    col = edge_index[1].astype(jnp.int32)
    w = weight.astype(f32)
    pad = E_PAD - E
    pidx = (jnp.arange(pad, dtype=jnp.int32) * 131) % N
    rowp = jnp.concatenate([row, pidx])
    colp = jnp.concatenate([col, pidx])
    wp = jnp.concatenate([w, jnp.zeros((pad,), f32)])
    wfix = (wp * WSCALE).astype(jnp.int32)
    epk = (jnp.stack([rowp, colp, wfix], axis=0)
           .reshape(3, E_PAD // CH, CH).transpose(1, 0, 2))

    deg_kernel, prop_kernel = _sc_kernels()
    deg2 = deg_kernel(colp, wp)[:, :N].reshape(NC, N, 1)

    Ws = (W1, W2, W3)
    bs = (jnp.broadcast_to(b1, (8, D)), jnp.broadcast_to(b2, (8, D)),
          jnp.broadcast_to(b3, (8, D)))
    dinv, g, s = _prep(x, deg2, W1[0])
    out = None
    for l in range(3):
        for k in range(1, 4):
            acc2 = prop_kernel(g, epk)
            if k < 3:
                s, g = _step(acc2, dinv, s, Ws[l][k])
            elif l < 2:
                g, s = _layerend(acc2, dinv, s, Ws[l][3], bs[l], Ws[l + 1][0])
            else:
                out = _final(acc2, dinv, s, Ws[2][3], bs[2])
    return out
